# baseline (speedup 1.0000x reference)
import functools

import jax
import jax.numpy as jnp
from jax import lax
from jax.experimental import pallas as pl
from jax.experimental.pallas import tpu as pltpu
from jax.experimental.pallas import tpu_sc as plsc

B = 16384
D = 64
NC = 2
NS = 16
NW = NC * NS
BPW = B // NW
CH = 128
NCH = BPW // CH

_mesh = plsc.VectorSubcoreMesh(
    core_axis_name="c", subcore_axis_name="s", num_cores=NC, num_subcores=NS
)


@functools.partial(
    pl.kernel,
    out_type=(
        jax.ShapeDtypeStruct((B, D), jnp.float32),
        jax.ShapeDtypeStruct((B, D), jnp.float32),
        jax.ShapeDtypeStruct((B, D), jnp.float32),
    ),
    mesh=_mesh,
    compiler_params=pltpu.CompilerParams(use_tc_tiling_on_sc=False),
    scratch_types=[
        pltpu.VMEM((NCH, CH), jnp.int32),
        pltpu.VMEM((NCH, CH), jnp.int32),
        pltpu.VMEM((NCH, CH), jnp.int32),
        pltpu.VMEM((BPW, D), jnp.float32),
        pltpu.VMEM((BPW, D), jnp.float32),
        pltpu.VMEM((BPW, D), jnp.float32),
        pltpu.SemaphoreType.DMA,
    ],
)
def _sc_gather(head_hbm, rel_hbm, tail_hbm, etab, rtab,
               out_h, out_r, out_t,
               hidx, ridx, tidx, hrow, rrow, trow, sem):
    wid = lax.axis_index("s") * NC + lax.axis_index("c")
    row0 = wid * NCH
    pltpu.sync_copy(head_hbm.at[pl.ds(row0, NCH)], hidx)
    pltpu.sync_copy(rel_hbm.at[pl.ds(row0, NCH)], ridx)
    pltpu.sync_copy(tail_hbm.at[pl.ds(row0, NCH)], tidx)
    copies = []
    for j in range(NCH):
        dst = pl.ds(j * CH, CH)
        copies.append(pltpu.async_copy(etab.at[hidx.at[j]], hrow.at[dst], sem))
        copies.append(pltpu.async_copy(rtab.at[ridx.at[j]], rrow.at[dst], sem))
        copies.append(pltpu.async_copy(etab.at[tidx.at[j]], trow.at[dst], sem))
    for c in copies:
        c.wait()
    base = wid * BPW
    pltpu.sync_copy(hrow, out_h.at[pl.ds(base, BPW)])
    pltpu.sync_copy(rrow, out_r.at[pl.ds(base, BPW)])
    pltpu.sync_copy(trow, out_t.at[pl.ds(base, BPW)])


def kernel(head, relation, tail, entity_table, relation_table):
    h = head.astype(jnp.int32).reshape(B // CH, CH)
    r = relation.astype(jnp.int32).reshape(B // CH, CH)
    t = tail.astype(jnp.int32).reshape(B // CH, CH)
    return _sc_gather(h, r, t, entity_table, relation_table)


# speedup vs baseline: 1.6613x; 1.6613x over previous
import functools

import jax
import jax.numpy as jnp
from jax import lax
from jax.experimental import pallas as pl
from jax.experimental.pallas import tpu as pltpu
from jax.experimental.pallas import tpu_sc as plsc

B = 16384
D = 64
NC = 2
NS = 16
NW = NC * NS
BPW = B // NW
G = 16

_mesh = plsc.VectorSubcoreMesh(
    core_axis_name="c", subcore_axis_name="s", num_cores=NC, num_subcores=NS
)


def _lookup(idx_hbm, tab, out, base, idxb, rows, sem):
    pltpu.sync_copy(idx_hbm.at[pl.ds(base, BPW)], idxb)

    def group(g, carry):
        svec = idxb[pl.ds(g * G, G)]
        for r in range(G):
            i = svec[r]
            pltpu.async_copy(
                tab.at[pl.ds(i, 1)],
                rows.at[pl.ds(g * G + r, 1)],
                sem,
            )
        return carry

    lax.fori_loop(0, BPW // G, group, 0, unroll=False)
    pltpu.make_async_copy(tab.at[pl.ds(0, BPW)], rows, sem).wait()
    pltpu.sync_copy(rows, out.at[pl.ds(base, BPW)])


@functools.partial(
    pl.kernel,
    out_type=(
        jax.ShapeDtypeStruct((B, D), jnp.float32),
        jax.ShapeDtypeStruct((B, D), jnp.float32),
        jax.ShapeDtypeStruct((B, D), jnp.float32),
    ),
    mesh=_mesh,
    scratch_types=[
        pltpu.VMEM((BPW,), jnp.int32),
        pltpu.VMEM((BPW, D), jnp.float32),
        pltpu.SemaphoreType.DMA,
    ],
)
def _sc_gather(head_hbm, rel_hbm, tail_hbm, etab, rtab,
               out_h, out_r, out_t,
               idxb, rows, sem):
    wid = lax.axis_index("s") * NC + lax.axis_index("c")
    base = wid * BPW
    _lookup(head_hbm, etab, out_h, base, idxb, rows, sem)
    _lookup(tail_hbm, etab, out_t, base, idxb, rows, sem)
    _lookup(rel_hbm, rtab, out_r, base, idxb, rows, sem)


def kernel(head, relation, tail, entity_table, relation_table):
    h = head.astype(jnp.int32)
    r = relation.astype(jnp.int32)
    t = tail.astype(jnp.int32)
    return _sc_gather(h, r, t, entity_table, relation_table)
